# baseline (speedup 1.0000x reference)
import functools

import jax
import jax.numpy as jnp
from jax import lax
from jax.experimental import pallas as pl
from jax.experimental.pallas import tpu as pltpu, tpu_sc as plsc

N = 10000
NP = 10240
D = 128
E = 320000
G = 64
L = 64

NC = 2
NS = 16
NW = NC * NS
EPT = E // NW
KC = 64
NCHUNK = 160
EPTP = KC * NCHUNK
NBUF = 4
NSUP = NCHUNK // NBUF
RPT = NP // NS

_mesh = plsc.VectorSubcoreMesh(core_axis_name="c", subcore_axis_name="s")


@functools.partial(
    pl.kernel,
    out_type=jax.ShapeDtypeStruct((NC, NP, D), jnp.float32),
    mesh=_mesh,
    scratch_types=[
        pltpu.VMEM((2, NBUF, 2, KC), jnp.int32),
        pltpu.VMEM((NBUF, KC, D), jnp.float32),
        pltpu.VMEM_SHARED((NP, D), jnp.float32),
        [pltpu.SemaphoreType.DMA] * (2 + 2 * NBUF),
    ],
)
def _sc_aggregate(x_hbm, idx_hbm, out_hbm, idxb, rows, agg_sh, sems):
    c = lax.axis_index("c")
    s = lax.axis_index("s")
    wid = s * NC + c
    r0 = s * RPT

    def idx_issue(j, bank):
        pltpu.async_copy(idx_hbm.at[wid].at[pl.ds(j * NBUF, NBUF)],
                         idxb.at[bank], sems[bank])

    def idx_wait(bank):
        pltpu.make_async_copy(idx_hbm.at[0].at[pl.ds(0, NBUF)],
                              idxb.at[bank], sems[bank]).wait()

    def gather_issue(b, bank):
        pltpu.async_copy(x_hbm.at[idxb.at[bank, b, 0]], rows.at[b],
                         sems[2 + b])

    def gather_wait(b):
        pltpu.make_async_copy(x_hbm.at[pl.ds(0, KC)], rows.at[b],
                              sems[2 + b]).wait()

    def scatter_issue(b, bank):
        pltpu.async_copy(rows.at[b], agg_sh.at[idxb.at[bank, b, 1]],
                         sems[2 + NBUF + b], add=True)

    def scatter_wait(b):
        pltpu.make_async_copy(rows.at[b], agg_sh.at[pl.ds(0, KC)],
                              sems[2 + NBUF + b]).wait()

    pltpu.sync_copy(x_hbm.at[pl.ds(r0, RPT)], agg_sh.at[pl.ds(r0, RPT)])
    idx_issue(0, 0)
    idx_wait(0)
    for b in range(NBUF):
        gather_issue(b, 0)
    idx_issue(1, 1)

    plsc.subcore_barrier()

    def half(j, bank, issue_next_idx, issue_next_gather):
        for b in range(NBUF):
            gather_wait(b)
            scatter_issue(b, bank)
        for b in range(NBUF):
            scatter_wait(b)
        if issue_next_idx:
            idx_issue(j + 2, bank)
        if issue_next_gather:
            idx_wait(1 - bank)
            for b in range(NBUF):
                gather_issue(b, 1 - bank)

    def body(jp, carry):
        half(2 * jp, 0, True, True)
        half(2 * jp + 1, 1, True, True)
        return carry

    lax.fori_loop(0, NSUP // 2 - 1, body, 0)
    half(NSUP - 2, 0, False, True)
    half(NSUP - 1, 1, False, False)

    plsc.subcore_barrier()
    pltpu.sync_copy(agg_sh.at[pl.ds(r0, RPT)], out_hbm.at[c].at[pl.ds(r0, RPT)])


R = 640
NBLK = NP // R
_BN_S = 1.0 / (1.0 + 1e-5) ** 0.5


def _mlp_body(p_ref, x_ref, w1_ref, b1_ref, g_ref, be_ref, w2_ref, b2_ref, out_ref):
    h = p_ref[0] + p_ref[1] - x_ref[...]
    u = jnp.dot(h, w1_ref[...], preferred_element_type=jnp.float32) + b1_ref[...]
    u = jnp.where(u >= 0, u, 0.2 * u)
    u = u * (g_ref[...] * _BN_S) + be_ref[...]
    v = jnp.dot(u, w2_ref[...], preferred_element_type=jnp.float32) + b2_ref[...]
    out_ref[...] = jnp.where(v >= 0, v, 0.2 * v)


_row_spec = pl.BlockSpec((R, D), lambda i: (i, 0))
_pair_spec = pl.BlockSpec((NC, R, D), lambda i: (0, i, 0))
_w_spec = pl.BlockSpec((D, D), lambda i: (0, 0))
_v_spec = pl.BlockSpec((1, D), lambda i: (0, 0))


def _mlp(p, x, w1, b1, g, be, w2, b2):
    return pl.pallas_call(
        _mlp_body,
        grid=(NBLK,),
        in_specs=[_pair_spec, _row_spec, _w_spec, _v_spec, _v_spec, _v_spec,
                  _w_spec, _v_spec],
        out_specs=_row_spec,
        out_shape=jax.ShapeDtypeStruct((NP, D), jnp.float32),
    )(p, x, w1, b1.reshape(1, D), g.reshape(1, D),
      be.reshape(1, D), w2, b2.reshape(1, D))


def _mlp_pool_body(p_ref, x_ref, batch_ref, w1_ref, b1_ref, g_ref, be_ref,
                   w2_ref, b2_ref, bng_ref, bnb_ref, fcw_ref, fcb_ref,
                   out_ref, acc_ref):
    i = pl.program_id(0)
    h = p_ref[0] + p_ref[1] - x_ref[...]
    u = jnp.dot(h, w1_ref[...], preferred_element_type=jnp.float32) + b1_ref[...]
    u = jnp.where(u >= 0, u, 0.2 * u)
    u = u * (g_ref[...] * _BN_S) + be_ref[...]
    v = jnp.dot(u, w2_ref[...], preferred_element_type=jnp.float32) + b2_ref[...]
    v = jnp.where(v >= 0, v, 0.2 * v)

    b = batch_ref[0, 0, :]
    oh = (b[:, None] == lax.broadcasted_iota(jnp.int32, (R, G), 1)).astype(jnp.float32)
    part = lax.dot_general(oh, v, (((0,), (0,)), ((), ())),
                           preferred_element_type=jnp.float32)

    @pl.when(i == 0)
    def _():
        acc_ref[...] = jnp.zeros_like(acc_ref)

    acc_ref[...] += part

    @pl.when(i == NBLK - 1)
    def _():
        pooled = acc_ref[...] * (bng_ref[...] * _BN_S) + bnb_ref[...]
        out_ref[...] = (jnp.dot(pooled, fcw_ref[...],
                                preferred_element_type=jnp.float32)
                        + fcb_ref[...])


def _mlp_pool(p, x, batch3, w1, b1, g, be, w2, b2, bng, bnb, fcw, fcb):
    return pl.pallas_call(
        _mlp_pool_body,
        grid=(NBLK,),
        in_specs=[_pair_spec, _row_spec,
                  pl.BlockSpec((1, 1, R), lambda i: (i, 0, 0)),
                  _w_spec, _v_spec, _v_spec, _v_spec, _w_spec, _v_spec,
                  pl.BlockSpec((1, D), lambda i: (0, 0)),
                  pl.BlockSpec((1, D), lambda i: (0, 0)),
                  pl.BlockSpec((D, L), lambda i: (0, 0)),
                  pl.BlockSpec((1, L), lambda i: (0, 0))],
        out_specs=pl.BlockSpec((G, L), lambda i: (0, 0)),
        out_shape=jax.ShapeDtypeStruct((G, L), jnp.float32),
        scratch_shapes=[pltpu.VMEM((G, D), jnp.float32)],
    )(p, x, batch3, w1, b1.reshape(1, D), g.reshape(1, D),
      be.reshape(1, D), w2, b2.reshape(1, D),
      bng.reshape(1, D), bnb.reshape(1, D), fcw, fcb.reshape(1, L))


def kernel(x, edge_index, batch, c0_w1, c0_b1, c0_g, c0_be, c0_w2, c0_b2,
           c1_w1, c1_b1, c1_g, c1_be, c1_w2, c1_b2,
           c2_w1, c2_b1, c2_g, c2_be, c2_w2, c2_b2,
           bn_g, bn_b, fc_w, fc_b):
    pad_e = EPTP - EPT
    src = jnp.pad(edge_index[0].reshape(NW, EPT), ((0, 0), (0, pad_e))
                  ).reshape(NW, NCHUNK, 1, KC)
    dst = jnp.pad(edge_index[1].reshape(NW, EPT), ((0, 0), (0, pad_e)),
                  constant_values=NP - 1).reshape(NW, NCHUNK, 1, KC)
    idx = jnp.concatenate([src, dst], axis=2)
    xp = jnp.pad(x, ((0, NP - N), (0, 0)))
    batch3 = jnp.pad(batch, (0, NP - N), constant_values=G).reshape(NBLK, 1, R)

    p = _sc_aggregate(xp, idx)
    h = _mlp(p, xp, c0_w1, c0_b1, c0_g, c0_be, c0_w2, c0_b2)
    p = _sc_aggregate(h, idx)
    h = _mlp(p, h, c1_w1, c1_b1, c1_g, c1_be, c1_w2, c1_b2)
    p = _sc_aggregate(h, idx)
    out = _mlp_pool(p, h, batch3, c2_w1, c2_b1, c2_g, c2_be, c2_w2, c2_b2,
                    bn_g, bn_b, fc_w, fc_b)
    return out


# speedup vs baseline: 1.0112x; 1.0112x over previous
import functools

import jax
import jax.numpy as jnp
from jax import lax
from jax.experimental import pallas as pl
from jax.experimental.pallas import tpu as pltpu, tpu_sc as plsc

N = 10000
NP = 10240
D = 128
E = 320000
G = 64
L = 64

NC = 2
NS = 16
NW = NC * NS
EPT = E // NW
KC = 80
NCHUNK = 128
EPTP = KC * NCHUNK
NBUF = 4
NSUP = NCHUNK // NBUF
RPT = NP // NS

_mesh = plsc.VectorSubcoreMesh(core_axis_name="c", subcore_axis_name="s")


@functools.partial(
    pl.kernel,
    out_type=jax.ShapeDtypeStruct((NC, NP, D), jnp.float32),
    mesh=_mesh,
    scratch_types=[
        pltpu.VMEM((2, NBUF, 2, KC), jnp.int32),
        pltpu.VMEM((NBUF, KC, D), jnp.float32),
        pltpu.VMEM_SHARED((NP, D), jnp.float32),
        [pltpu.SemaphoreType.DMA] * (2 + 2 * NBUF),
    ],
)
def _sc_aggregate(x_hbm, idx_hbm, out_hbm, idxb, rows, agg_sh, sems):
    c = lax.axis_index("c")
    s = lax.axis_index("s")
    wid = s * NC + c
    r0 = s * RPT

    def idx_issue(j, bank):
        pltpu.async_copy(idx_hbm.at[wid].at[pl.ds(j * NBUF, NBUF)],
                         idxb.at[bank], sems[bank])

    def idx_wait(bank):
        pltpu.make_async_copy(idx_hbm.at[0].at[pl.ds(0, NBUF)],
                              idxb.at[bank], sems[bank]).wait()

    def gather_issue(b, bank):
        pltpu.async_copy(x_hbm.at[idxb.at[bank, b, 0]], rows.at[b],
                         sems[2 + b])

    def gather_wait(b):
        pltpu.make_async_copy(x_hbm.at[pl.ds(0, KC)], rows.at[b],
                              sems[2 + b]).wait()

    def scatter_issue(b, bank):
        pltpu.async_copy(rows.at[b], agg_sh.at[idxb.at[bank, b, 1]],
                         sems[2 + NBUF + b], add=True)

    def scatter_wait(b):
        pltpu.make_async_copy(rows.at[b], agg_sh.at[pl.ds(0, KC)],
                              sems[2 + NBUF + b]).wait()

    pltpu.sync_copy(x_hbm.at[pl.ds(r0, RPT)], agg_sh.at[pl.ds(r0, RPT)])
    idx_issue(0, 0)
    idx_wait(0)
    for b in range(NBUF):
        gather_issue(b, 0)
    idx_issue(1, 1)

    plsc.subcore_barrier()

    def half(j, bank, issue_next_idx, issue_next_gather):
        for b in range(NBUF):
            gather_wait(b)
            scatter_issue(b, bank)
        for b in range(NBUF):
            scatter_wait(b)
        if issue_next_idx:
            idx_issue(j + 2, bank)
        if issue_next_gather:
            idx_wait(1 - bank)
            for b in range(NBUF):
                gather_issue(b, 1 - bank)

    def body(jp, carry):
        half(2 * jp, 0, True, True)
        half(2 * jp + 1, 1, True, True)
        return carry

    lax.fori_loop(0, NSUP // 2 - 1, body, 0)
    half(NSUP - 2, 0, False, True)
    half(NSUP - 1, 1, False, False)

    plsc.subcore_barrier()
    pltpu.sync_copy(agg_sh.at[pl.ds(r0, RPT)], out_hbm.at[c].at[pl.ds(r0, RPT)])


R = 640
NBLK = NP // R
_BN_S = 1.0 / (1.0 + 1e-5) ** 0.5


def _mlp_body(p_ref, x_ref, w1_ref, b1_ref, g_ref, be_ref, w2_ref, b2_ref, out_ref):
    h = p_ref[0] + p_ref[1] - x_ref[...]
    u = jnp.dot(h, w1_ref[...], preferred_element_type=jnp.float32) + b1_ref[...]
    u = jnp.where(u >= 0, u, 0.2 * u)
    u = u * (g_ref[...] * _BN_S) + be_ref[...]
    v = jnp.dot(u, w2_ref[...], preferred_element_type=jnp.float32) + b2_ref[...]
    out_ref[...] = jnp.where(v >= 0, v, 0.2 * v)


_row_spec = pl.BlockSpec((R, D), lambda i: (i, 0))
_pair_spec = pl.BlockSpec((NC, R, D), lambda i: (0, i, 0))
_w_spec = pl.BlockSpec((D, D), lambda i: (0, 0))
_v_spec = pl.BlockSpec((1, D), lambda i: (0, 0))


def _mlp(p, x, w1, b1, g, be, w2, b2):
    return pl.pallas_call(
        _mlp_body,
        grid=(NBLK,),
        in_specs=[_pair_spec, _row_spec, _w_spec, _v_spec, _v_spec, _v_spec,
                  _w_spec, _v_spec],
        out_specs=_row_spec,
        out_shape=jax.ShapeDtypeStruct((NP, D), jnp.float32),
    )(p, x, w1, b1.reshape(1, D), g.reshape(1, D),
      be.reshape(1, D), w2, b2.reshape(1, D))


def _mlp_pool_body(p_ref, x_ref, batch_ref, w1_ref, b1_ref, g_ref, be_ref,
                   w2_ref, b2_ref, bng_ref, bnb_ref, fcw_ref, fcb_ref,
                   out_ref, acc_ref):
    i = pl.program_id(0)
    h = p_ref[0] + p_ref[1] - x_ref[...]
    u = jnp.dot(h, w1_ref[...], preferred_element_type=jnp.float32) + b1_ref[...]
    u = jnp.where(u >= 0, u, 0.2 * u)
    u = u * (g_ref[...] * _BN_S) + be_ref[...]
    v = jnp.dot(u, w2_ref[...], preferred_element_type=jnp.float32) + b2_ref[...]
    v = jnp.where(v >= 0, v, 0.2 * v)

    b = batch_ref[0, 0, :]
    oh = (b[:, None] == lax.broadcasted_iota(jnp.int32, (R, G), 1)).astype(jnp.float32)
    part = lax.dot_general(oh, v, (((0,), (0,)), ((), ())),
                           preferred_element_type=jnp.float32)

    @pl.when(i == 0)
    def _():
        acc_ref[...] = jnp.zeros_like(acc_ref)

    acc_ref[...] += part

    @pl.when(i == NBLK - 1)
    def _():
        pooled = acc_ref[...] * (bng_ref[...] * _BN_S) + bnb_ref[...]
        out_ref[...] = (jnp.dot(pooled, fcw_ref[...],
                                preferred_element_type=jnp.float32)
                        + fcb_ref[...])


def _mlp_pool(p, x, batch3, w1, b1, g, be, w2, b2, bng, bnb, fcw, fcb):
    return pl.pallas_call(
        _mlp_pool_body,
        grid=(NBLK,),
        in_specs=[_pair_spec, _row_spec,
                  pl.BlockSpec((1, 1, R), lambda i: (i, 0, 0)),
                  _w_spec, _v_spec, _v_spec, _v_spec, _w_spec, _v_spec,
                  pl.BlockSpec((1, D), lambda i: (0, 0)),
                  pl.BlockSpec((1, D), lambda i: (0, 0)),
                  pl.BlockSpec((D, L), lambda i: (0, 0)),
                  pl.BlockSpec((1, L), lambda i: (0, 0))],
        out_specs=pl.BlockSpec((G, L), lambda i: (0, 0)),
        out_shape=jax.ShapeDtypeStruct((G, L), jnp.float32),
        scratch_shapes=[pltpu.VMEM((G, D), jnp.float32)],
    )(p, x, batch3, w1, b1.reshape(1, D), g.reshape(1, D),
      be.reshape(1, D), w2, b2.reshape(1, D),
      bng.reshape(1, D), bnb.reshape(1, D), fcw, fcb.reshape(1, L))


def kernel(x, edge_index, batch, c0_w1, c0_b1, c0_g, c0_be, c0_w2, c0_b2,
           c1_w1, c1_b1, c1_g, c1_be, c1_w2, c1_b2,
           c2_w1, c2_b1, c2_g, c2_be, c2_w2, c2_b2,
           bn_g, bn_b, fc_w, fc_b):
    pad_e = EPTP - EPT
    src = jnp.pad(edge_index[0].reshape(NW, EPT), ((0, 0), (0, pad_e))
                  ).reshape(NW, NCHUNK, 1, KC)
    dst = jnp.pad(edge_index[1].reshape(NW, EPT), ((0, 0), (0, pad_e)),
                  constant_values=NP - 1).reshape(NW, NCHUNK, 1, KC)
    idx = jnp.concatenate([src, dst], axis=2)
    xp = jnp.pad(x, ((0, NP - N), (0, 0)))
    batch3 = jnp.pad(batch, (0, NP - N), constant_values=G).reshape(NBLK, 1, R)

    p = _sc_aggregate(xp, idx)
    h = _mlp(p, xp, c0_w1, c0_b1, c0_g, c0_be, c0_w2, c0_b2)
    p = _sc_aggregate(h, idx)
    h = _mlp(p, h, c1_w1, c1_b1, c1_g, c1_be, c1_w2, c1_b2)
    p = _sc_aggregate(h, idx)
    out = _mlp_pool(p, h, batch3, c2_w1, c2_b1, c2_g, c2_be, c2_w2, c2_b2,
                    bn_g, bn_b, fc_w, fc_b)
    return out


# speedup vs baseline: 1.7707x; 1.7511x over previous
import functools

import jax
import jax.numpy as jnp
from jax import lax
from jax.experimental import pallas as pl
from jax.experimental.pallas import tpu as pltpu, tpu_sc as plsc

N = 10000
NP = 10240
D = 128
E = 320000
G = 64
L = 64

NC = 2
NS = 16
NW = NC * NS
EPT = E // NW
KC = 112
NCHUNK = 90
EPTP = KC * NCHUNK
NBUF = 3
NSUP = NCHUNK // NBUF
RPT = NP // NS

_mesh = plsc.VectorSubcoreMesh(core_axis_name="c", subcore_axis_name="s")


@functools.partial(
    pl.kernel,
    out_type=jax.ShapeDtypeStruct((NC, NP, D), jnp.float32),
    mesh=_mesh,
    scratch_types=[
        pltpu.VMEM((2, NBUF, 2, KC), jnp.int32),
        pltpu.VMEM((NBUF, KC, D), jnp.float32),
        pltpu.VMEM_SHARED((NP, D), jnp.float32),
        [pltpu.SemaphoreType.DMA] * (2 + 2 * NBUF),
    ],
)
def _sc_aggregate(x_hbm, idx_hbm, out_hbm, idxb, rows, agg_sh, sems):
    c = lax.axis_index("c")
    s = lax.axis_index("s")
    wid = s * NC + c
    r0 = s * RPT

    def idx_issue(j, bank):
        pltpu.async_copy(idx_hbm.at[wid].at[pl.ds(j * NBUF, NBUF)],
                         idxb.at[bank], sems[bank])

    def idx_wait(bank):
        pltpu.make_async_copy(idx_hbm.at[0].at[pl.ds(0, NBUF)],
                              idxb.at[bank], sems[bank]).wait()

    def gather_issue(b, bank):
        pltpu.async_copy(x_hbm.at[idxb.at[bank, b, 0]], rows.at[b],
                         sems[2 + b])

    def gather_wait(b):
        pltpu.make_async_copy(x_hbm.at[pl.ds(0, KC)], rows.at[b],
                              sems[2 + b]).wait()

    def scatter_issue(b, bank):
        pltpu.async_copy(rows.at[b], agg_sh.at[idxb.at[bank, b, 1]],
                         sems[2 + NBUF + b], add=True)

    def scatter_wait(b):
        pltpu.make_async_copy(rows.at[b], agg_sh.at[pl.ds(0, KC)],
                              sems[2 + NBUF + b]).wait()

    pltpu.sync_copy(x_hbm.at[pl.ds(r0, RPT)], agg_sh.at[pl.ds(r0, RPT)])
    idx_issue(0, 0)
    idx_wait(0)
    for b in range(NBUF):
        gather_issue(b, 0)
    idx_issue(1, 1)

    plsc.subcore_barrier()

    def half(j, bank, issue_next_idx, issue_next_gather):
        for b in range(NBUF):
            gather_wait(b)
            scatter_issue(b, bank)
        for b in range(NBUF):
            scatter_wait(b)
        if issue_next_idx:
            idx_issue(j + 2, bank)
        if issue_next_gather:
            idx_wait(1 - bank)
            for b in range(NBUF):
                gather_issue(b, 1 - bank)

    def body(jp, carry):
        half(2 * jp, 0, True, True)
        half(2 * jp + 1, 1, True, True)
        return carry

    lax.fori_loop(0, NSUP // 2 - 1, body, 0)
    half(NSUP - 2, 0, False, True)
    half(NSUP - 1, 1, False, False)

    plsc.subcore_barrier()
    pltpu.sync_copy(agg_sh.at[pl.ds(r0, RPT)], out_hbm.at[c].at[pl.ds(r0, RPT)])


R = 640
NBLK = NP // R
_BN_S = 1.0 / (1.0 + 1e-5) ** 0.5


def _mlp_body(p_ref, x_ref, w1_ref, b1_ref, g_ref, be_ref, w2_ref, b2_ref, out_ref):
    h = p_ref[0] + p_ref[1] - x_ref[...]
    u = jnp.dot(h, w1_ref[...], preferred_element_type=jnp.float32) + b1_ref[...]
    u = jnp.where(u >= 0, u, 0.2 * u)
    u = u * (g_ref[...] * _BN_S) + be_ref[...]
    v = jnp.dot(u, w2_ref[...], preferred_element_type=jnp.float32) + b2_ref[...]
    out_ref[...] = jnp.where(v >= 0, v, 0.2 * v)


_row_spec = pl.BlockSpec((R, D), lambda i: (i, 0))
_pair_spec = pl.BlockSpec((NC, R, D), lambda i: (0, i, 0))
_w_spec = pl.BlockSpec((D, D), lambda i: (0, 0))
_v_spec = pl.BlockSpec((1, D), lambda i: (0, 0))


def _mlp(p, x, w1, b1, g, be, w2, b2):
    return pl.pallas_call(
        _mlp_body,
        grid=(NBLK,),
        in_specs=[_pair_spec, _row_spec, _w_spec, _v_spec, _v_spec, _v_spec,
                  _w_spec, _v_spec],
        out_specs=_row_spec,
        out_shape=jax.ShapeDtypeStruct((NP, D), jnp.float32),
    )(p, x, w1, b1.reshape(1, D), g.reshape(1, D),
      be.reshape(1, D), w2, b2.reshape(1, D))


def _mlp_pool_body(p_ref, x_ref, batch_ref, w1_ref, b1_ref, g_ref, be_ref,
                   w2_ref, b2_ref, bng_ref, bnb_ref, fcw_ref, fcb_ref,
                   out_ref, acc_ref):
    i = pl.program_id(0)
    h = p_ref[0] + p_ref[1] - x_ref[...]
    u = jnp.dot(h, w1_ref[...], preferred_element_type=jnp.float32) + b1_ref[...]
    u = jnp.where(u >= 0, u, 0.2 * u)
    u = u * (g_ref[...] * _BN_S) + be_ref[...]
    v = jnp.dot(u, w2_ref[...], preferred_element_type=jnp.float32) + b2_ref[...]
    v = jnp.where(v >= 0, v, 0.2 * v)

    b = batch_ref[0, 0, :]
    oh = (b[:, None] == lax.broadcasted_iota(jnp.int32, (R, G), 1)).astype(jnp.float32)
    part = lax.dot_general(oh, v, (((0,), (0,)), ((), ())),
                           preferred_element_type=jnp.float32)

    @pl.when(i == 0)
    def _():
        acc_ref[...] = jnp.zeros_like(acc_ref)

    acc_ref[...] += part

    @pl.when(i == NBLK - 1)
    def _():
        pooled = acc_ref[...] * (bng_ref[...] * _BN_S) + bnb_ref[...]
        out_ref[...] = (jnp.dot(pooled, fcw_ref[...],
                                preferred_element_type=jnp.float32)
                        + fcb_ref[...])


def _mlp_pool(p, x, batch3, w1, b1, g, be, w2, b2, bng, bnb, fcw, fcb):
    return pl.pallas_call(
        _mlp_pool_body,
        grid=(NBLK,),
        in_specs=[_pair_spec, _row_spec,
                  pl.BlockSpec((1, 1, R), lambda i: (i, 0, 0)),
                  _w_spec, _v_spec, _v_spec, _v_spec, _w_spec, _v_spec,
                  pl.BlockSpec((1, D), lambda i: (0, 0)),
                  pl.BlockSpec((1, D), lambda i: (0, 0)),
                  pl.BlockSpec((D, L), lambda i: (0, 0)),
                  pl.BlockSpec((1, L), lambda i: (0, 0))],
        out_specs=pl.BlockSpec((G, L), lambda i: (0, 0)),
        out_shape=jax.ShapeDtypeStruct((G, L), jnp.float32),
        scratch_shapes=[pltpu.VMEM((G, D), jnp.float32)],
    )(p, x, batch3, w1, b1.reshape(1, D), g.reshape(1, D),
      be.reshape(1, D), w2, b2.reshape(1, D),
      bng.reshape(1, D), bnb.reshape(1, D), fcw, fcb.reshape(1, L))


def kernel(x, edge_index, batch, c0_w1, c0_b1, c0_g, c0_be, c0_w2, c0_b2,
           c1_w1, c1_b1, c1_g, c1_be, c1_w2, c1_b2,
           c2_w1, c2_b1, c2_g, c2_be, c2_w2, c2_b2,
           bn_g, bn_b, fc_w, fc_b):
    pad_e = EPTP - EPT
    src = jnp.pad(edge_index[0].reshape(NW, EPT), ((0, 0), (0, pad_e))
                  ).reshape(NW, NCHUNK, 1, KC)
    dst = jnp.pad(edge_index[1].reshape(NW, EPT), ((0, 0), (0, pad_e)),
                  constant_values=NP - 1).reshape(NW, NCHUNK, 1, KC)
    idx = jnp.concatenate([src, dst], axis=2)
    xp = jnp.pad(x, ((0, NP - N), (0, 0)))
    batch3 = jnp.pad(batch, (0, NP - N), constant_values=G).reshape(NBLK, 1, R)

    p = _sc_aggregate(xp, idx)
    h = _mlp(p, xp, c0_w1, c0_b1, c0_g, c0_be, c0_w2, c0_b2)
    p = _sc_aggregate(h, idx)
    h = _mlp(p, h, c1_w1, c1_b1, c1_g, c1_be, c1_w2, c1_b2)
    p = _sc_aggregate(h, idx)
    out = _mlp_pool(p, h, batch3, c2_w1, c2_b1, c2_g, c2_be, c2_w2, c2_b2,
                    bn_g, bn_b, fc_w, fc_b)
    return out


# speedup vs baseline: 1.7750x; 1.0025x over previous
import functools

import jax
import jax.numpy as jnp
from jax import lax
from jax.experimental import pallas as pl
from jax.experimental.pallas import tpu as pltpu, tpu_sc as plsc

N = 10000
NP = 10240
D = 128
E = 320000
G = 64
L = 64

NC = 2
NS = 16
NW = NC * NS
EPT = E // NW
KC = 112
NCHUNK = 90
EPTP = KC * NCHUNK
NBUF = 3
NSUP = NCHUNK // NBUF
RPT = NP // NS

_mesh = plsc.VectorSubcoreMesh(core_axis_name="c", subcore_axis_name="s")


@functools.partial(
    pl.kernel,
    out_type=jax.ShapeDtypeStruct((NC, NP, D), jnp.float32),
    mesh=_mesh,
    scratch_types=[
        pltpu.VMEM((2, NBUF, 2, KC), jnp.int32),
        pltpu.VMEM((NBUF, KC, D), jnp.float32),
        pltpu.VMEM_SHARED((NP, D), jnp.float32),
        [pltpu.SemaphoreType.DMA] * (3 + 2 * NBUF),
    ],
)
def _sc_aggregate(x_hbm, idx_hbm, out_hbm, idxb, rows, agg_sh, sems):
    c = lax.axis_index("c")
    s = lax.axis_index("s")
    wid = s * NC + c
    r0 = s * RPT

    def idx_issue(j, bank):
        pltpu.async_copy(idx_hbm.at[wid].at[pl.ds(j * NBUF, NBUF)],
                         idxb.at[bank], sems[bank])

    def idx_wait(bank):
        pltpu.make_async_copy(idx_hbm.at[0].at[pl.ds(0, NBUF)],
                              idxb.at[bank], sems[bank]).wait()

    def gather_issue(b, bank):
        pltpu.async_copy(x_hbm.at[idxb.at[bank, b, 0]], rows.at[b],
                         sems[2 + b])

    def gather_wait(b):
        pltpu.make_async_copy(x_hbm.at[pl.ds(0, KC)], rows.at[b],
                              sems[2 + b]).wait()

    def scatter_issue(b, bank):
        pltpu.async_copy(rows.at[b], agg_sh.at[idxb.at[bank, b, 1]],
                         sems[2 + NBUF + b], add=True)

    def scatter_wait(b):
        pltpu.make_async_copy(rows.at[b], agg_sh.at[pl.ds(0, KC)],
                              sems[2 + NBUF + b]).wait()

    pltpu.async_copy(x_hbm.at[pl.ds(r0, RPT)], agg_sh.at[pl.ds(r0, RPT)],
                     sems[2 + 2 * NBUF])
    idx_issue(0, 0)
    idx_wait(0)
    for b in range(NBUF):
        gather_issue(b, 0)
    idx_issue(1, 1)

    pltpu.make_async_copy(x_hbm.at[pl.ds(r0, RPT)],
                          agg_sh.at[pl.ds(r0, RPT)],
                          sems[2 + 2 * NBUF]).wait()
    plsc.subcore_barrier()

    def half(j, bank, issue_next_idx, issue_next_gather):
        for b in range(NBUF):
            gather_wait(b)
            scatter_issue(b, bank)
        for b in range(NBUF):
            scatter_wait(b)
        if issue_next_idx:
            idx_issue(j + 2, bank)
        if issue_next_gather:
            idx_wait(1 - bank)
            for b in range(NBUF):
                gather_issue(b, 1 - bank)

    def body(jp, carry):
        half(2 * jp, 0, True, True)
        half(2 * jp + 1, 1, True, True)
        return carry

    lax.fori_loop(0, NSUP // 2 - 1, body, 0)
    half(NSUP - 2, 0, False, True)
    half(NSUP - 1, 1, False, False)

    plsc.subcore_barrier()
    pltpu.sync_copy(agg_sh.at[pl.ds(r0, RPT)], out_hbm.at[c].at[pl.ds(r0, RPT)])


R = 640
NBLK = NP // R
_BN_S = 1.0 / (1.0 + 1e-5) ** 0.5


def _mlp_body(p_ref, x_ref, w1_ref, b1_ref, g_ref, be_ref, w2_ref, b2_ref, out_ref):
    h = p_ref[0] + p_ref[1] - x_ref[...]
    u = jnp.dot(h, w1_ref[...], preferred_element_type=jnp.float32) + b1_ref[...]
    u = jnp.where(u >= 0, u, 0.2 * u)
    u = u * (g_ref[...] * _BN_S) + be_ref[...]
    v = jnp.dot(u, w2_ref[...], preferred_element_type=jnp.float32) + b2_ref[...]
    out_ref[...] = jnp.where(v >= 0, v, 0.2 * v)


_row_spec = pl.BlockSpec((R, D), lambda i: (i, 0))
_pair_spec = pl.BlockSpec((NC, R, D), lambda i: (0, i, 0))
_w_spec = pl.BlockSpec((D, D), lambda i: (0, 0))
_v_spec = pl.BlockSpec((1, D), lambda i: (0, 0))


def _mlp(p, x, w1, b1, g, be, w2, b2):
    return pl.pallas_call(
        _mlp_body,
        grid=(NBLK,),
        in_specs=[_pair_spec, _row_spec, _w_spec, _v_spec, _v_spec, _v_spec,
                  _w_spec, _v_spec],
        out_specs=_row_spec,
        out_shape=jax.ShapeDtypeStruct((NP, D), jnp.float32),
    )(p, x, w1, b1.reshape(1, D), g.reshape(1, D),
      be.reshape(1, D), w2, b2.reshape(1, D))


def _mlp_pool_body(p_ref, x_ref, batch_ref, w1_ref, b1_ref, g_ref, be_ref,
                   w2_ref, b2_ref, bng_ref, bnb_ref, fcw_ref, fcb_ref,
                   out_ref, acc_ref):
    i = pl.program_id(0)
    h = p_ref[0] + p_ref[1] - x_ref[...]
    u = jnp.dot(h, w1_ref[...], preferred_element_type=jnp.float32) + b1_ref[...]
    u = jnp.where(u >= 0, u, 0.2 * u)
    u = u * (g_ref[...] * _BN_S) + be_ref[...]
    v = jnp.dot(u, w2_ref[...], preferred_element_type=jnp.float32) + b2_ref[...]
    v = jnp.where(v >= 0, v, 0.2 * v)

    b = batch_ref[0, 0, :]
    oh = (b[:, None] == lax.broadcasted_iota(jnp.int32, (R, G), 1)).astype(jnp.float32)
    part = lax.dot_general(oh, v, (((0,), (0,)), ((), ())),
                           preferred_element_type=jnp.float32)

    @pl.when(i == 0)
    def _():
        acc_ref[...] = jnp.zeros_like(acc_ref)

    acc_ref[...] += part

    @pl.when(i == NBLK - 1)
    def _():
        pooled = acc_ref[...] * (bng_ref[...] * _BN_S) + bnb_ref[...]
        out_ref[...] = (jnp.dot(pooled, fcw_ref[...],
                                preferred_element_type=jnp.float32)
                        + fcb_ref[...])


def _mlp_pool(p, x, batch3, w1, b1, g, be, w2, b2, bng, bnb, fcw, fcb):
    return pl.pallas_call(
        _mlp_pool_body,
        grid=(NBLK,),
        in_specs=[_pair_spec, _row_spec,
                  pl.BlockSpec((1, 1, R), lambda i: (i, 0, 0)),
                  _w_spec, _v_spec, _v_spec, _v_spec, _w_spec, _v_spec,
                  pl.BlockSpec((1, D), lambda i: (0, 0)),
                  pl.BlockSpec((1, D), lambda i: (0, 0)),
                  pl.BlockSpec((D, L), lambda i: (0, 0)),
                  pl.BlockSpec((1, L), lambda i: (0, 0))],
        out_specs=pl.BlockSpec((G, L), lambda i: (0, 0)),
        out_shape=jax.ShapeDtypeStruct((G, L), jnp.float32),
        scratch_shapes=[pltpu.VMEM((G, D), jnp.float32)],
    )(p, x, batch3, w1, b1.reshape(1, D), g.reshape(1, D),
      be.reshape(1, D), w2, b2.reshape(1, D),
      bng.reshape(1, D), bnb.reshape(1, D), fcw, fcb.reshape(1, L))


def kernel(x, edge_index, batch, c0_w1, c0_b1, c0_g, c0_be, c0_w2, c0_b2,
           c1_w1, c1_b1, c1_g, c1_be, c1_w2, c1_b2,
           c2_w1, c2_b1, c2_g, c2_be, c2_w2, c2_b2,
           bn_g, bn_b, fc_w, fc_b):
    pad_e = EPTP - EPT
    src = jnp.pad(edge_index[0].reshape(NW, EPT), ((0, 0), (0, pad_e))
                  ).reshape(NW, NCHUNK, 1, KC)
    dst = jnp.pad(edge_index[1].reshape(NW, EPT), ((0, 0), (0, pad_e)),
                  constant_values=NP - 1).reshape(NW, NCHUNK, 1, KC)
    idx = jnp.concatenate([src, dst], axis=2)
    xp = jnp.pad(x, ((0, NP - N), (0, 0)))
    batch3 = jnp.pad(batch, (0, NP - N), constant_values=G).reshape(NBLK, 1, R)

    p = _sc_aggregate(xp, idx)
    h = _mlp(p, xp, c0_w1, c0_b1, c0_g, c0_be, c0_w2, c0_b2)
    p = _sc_aggregate(h, idx)
    h = _mlp(p, h, c1_w1, c1_b1, c1_g, c1_be, c1_w2, c1_b2)
    p = _sc_aggregate(h, idx)
    out = _mlp_pool(p, h, batch3, c2_w1, c2_b1, c2_g, c2_be, c2_w2, c2_b2,
                    bn_g, bn_b, fc_w, fc_b)
    return out
